# Initial kernel scaffold; baseline (speedup 1.0000x reference)
import jax
import jax.numpy as jnp
from jax.experimental import pallas as pl


def kernel(inputs):
    raise NotImplementedError("write your pallas kernel here")


# speedup vs baseline: 7.9066x; 7.9066x over previous
import functools

import jax
import jax.numpy as jnp
from jax import lax
from jax.experimental import pallas as pl
from jax.experimental.pallas import tpu as pltpu
from jax.experimental.pallas import tpu_sc as plsc

B, H, W, C = 4, 112, 112, 96
HD, WD = 2 * H, 2 * W
LANES = 16
CV = C // LANES
NC, NS = 2, 16
NW = NC * NS
UNITS_PER_IMG = H
UNITS = B * UNITS_PER_IMG
UPW = UNITS // NW

THIRD = 1.0 / 3.0
TWO_THIRD = 2.0 / 3.0


def _splat(val):
    return jnp.full((LANES,), val, jnp.float32)


def _col_mix(m_v, o_v):
    def jbody(j, carry):
        for cc in range(CV):
            s = pl.ds(cc * LANES, LANES)
            mj = m_v[j, s]
            mj1 = m_v[j + 1, s]
            o_v[2 * j + 1, s] = 0.75 * mj + 0.25 * mj1
            o_v[2 * j + 2, s] = 0.25 * mj + 0.75 * mj1
        return carry

    lax.fori_loop(1, W - 2, jbody, 0)

    for cc in range(CV):
        s = pl.ds(cc * LANES, LANES)
        m0 = m_v[0, s]
        m1 = m_v[1, s]
        o_v[0, s] = THIRD * m0
        o_v[1, s] = m0
        o_v[2, s] = THIRD * m0 + TWO_THIRD * m1
        mt0 = m_v[W - 2, s]
        mt1 = m_v[W - 1, s]
        o_v[WD - 3, s] = TWO_THIRD * mt0 + THIRD * mt1
        o_v[WD - 2, s] = mt1
        o_v[WD - 1, s] = THIRD * mt1


def _body(x_hbm, out_hbm, a_v, b_v, m1_v, m2_v, o1_v, o2_v):
    wid = lax.axis_index("s") * NC + lax.axis_index("c")

    def unit(t, carry):
        g = wid * UPW + t
        bb = g // UNITS_PER_IMG
        p = g - bb * UNITS_PER_IMG

        is_edge = p == UNITS_PER_IMG - 1
        is_lo = p == 0
        is_hi = p == UNITS_PER_IMG - 2

        ra = jnp.where(is_edge, 0, jnp.minimum(p, H - 2))
        rb = jnp.where(is_edge, H - 1, ra + 1)
        u1 = jnp.where(is_edge, 0, 2 * p + 1)
        u2 = jnp.where(is_edge, HD - 1, 2 * p + 2)

        def wsel(lo, hi, edge, default):
            r = jnp.where(is_lo, _splat(lo), _splat(default))
            r = jnp.where(is_hi, _splat(hi), r)
            return jnp.where(is_edge, _splat(edge), r)

        a1 = wsel(1.0, TWO_THIRD, THIRD, 0.75)
        b1 = wsel(0.0, THIRD, 0.0, 0.25)
        a2 = wsel(THIRD, 0.0, 0.0, 0.25)
        b2 = wsel(TWO_THIRD, 1.0, THIRD, 0.75)

        pltpu.sync_copy(x_hbm.at[bb, ra], a_v)
        pltpu.sync_copy(x_hbm.at[bb, rb], b_v)

        def hrow(w, c2):
            for cc in range(CV):
                s = pl.ds(cc * LANES, LANES)
                av = a_v[w, s]
                bv = b_v[w, s]
                m1_v[w, s] = a1 * av + b1 * bv
                m2_v[w, s] = a2 * av + b2 * bv
            return c2

        lax.fori_loop(0, W, hrow, 0)

        _col_mix(m1_v, o1_v)
        _col_mix(m2_v, o2_v)

        pltpu.sync_copy(o1_v, out_hbm.at[bb, u1])
        pltpu.sync_copy(o2_v, out_hbm.at[bb, u2])
        return carry

    lax.fori_loop(0, UPW, unit, 0)


@jax.jit
def _upsample(x):
    mesh = plsc.VectorSubcoreMesh(core_axis_name="c", subcore_axis_name="s")
    f = functools.partial(
        pl.kernel,
        mesh=mesh,
        out_type=jax.ShapeDtypeStruct((B, HD, WD, C), jnp.float32),
        scratch_types=[
            pltpu.VMEM((W, C), jnp.float32),
            pltpu.VMEM((W, C), jnp.float32),
            pltpu.VMEM((W, C), jnp.float32),
            pltpu.VMEM((W, C), jnp.float32),
            pltpu.VMEM((WD, C), jnp.float32),
            pltpu.VMEM((WD, C), jnp.float32),
        ],
    )(_body)
    return f(x)


def kernel(inputs):
    return _upsample(inputs)


# speedup vs baseline: 9.8725x; 1.2486x over previous
import functools

import jax
import jax.numpy as jnp
from jax import lax
from jax.experimental import pallas as pl
from jax.experimental.pallas import tpu as pltpu
from jax.experimental.pallas import tpu_sc as plsc

B, H, W, C = 4, 112, 112, 96
HD, WD = 2 * H, 2 * W
LANES = 16
CV = C // LANES
NC, NS = 2, 16
NW = NC * NS
UNITS_PER_IMG = H
UNITS = B * UNITS_PER_IMG
UPW = UNITS // NW

THIRD = 1.0 / 3.0
TWO_THIRD = 2.0 / 3.0


def _splat(val):
    return jnp.full((LANES,), val, jnp.float32)


def _col_mix(m_v, o_v):
    def jbody(j, carry):
        for cc in range(CV):
            s = pl.ds(cc * LANES, LANES)
            mj = m_v[j, s]
            mj1 = m_v[j + 1, s]
            o_v[2 * j + 1, s] = 0.75 * mj + 0.25 * mj1
            o_v[2 * j + 2, s] = 0.25 * mj + 0.75 * mj1
        return carry

    lax.fori_loop(1, W - 2, jbody, 0)

    for cc in range(CV):
        s = pl.ds(cc * LANES, LANES)
        m0 = m_v[0, s]
        m1 = m_v[1, s]
        o_v[0, s] = THIRD * m0
        o_v[1, s] = m0
        o_v[2, s] = THIRD * m0 + TWO_THIRD * m1
        mt0 = m_v[W - 2, s]
        mt1 = m_v[W - 1, s]
        o_v[WD - 3, s] = TWO_THIRD * mt0 + THIRD * mt1
        o_v[WD - 2, s] = mt1
        o_v[WD - 1, s] = THIRD * mt1


def _body(x_hbm, out_hbm, a0_v, b0_v, a1_v, b1_v, o1_v, o2_v,
          in_sem0, in_sem1, out_sem1, out_sem2):
    wid = lax.axis_index("s") * NC + lax.axis_index("c")

    def unit_rows(t):
        g = wid * UPW + t
        bb = g // UNITS_PER_IMG
        p = g - bb * UNITS_PER_IMG
        is_edge = p == UNITS_PER_IMG - 1
        ra = jnp.where(is_edge, 0, jnp.minimum(p, H - 2))
        rb = jnp.where(is_edge, H - 1, ra + 1)
        return bb, p, is_edge, ra, rb

    def start_in(t, av, bv, sem):
        bb, _, _, ra, rb = unit_rows(t)
        pltpu.async_copy(x_hbm.at[bb, ra], av, sem)
        pltpu.async_copy(x_hbm.at[bb, rb], bv, sem)

    def wait_in(av, bv, sem):
        pltpu.make_async_copy(x_hbm.at[0, 0], av, sem).wait()
        pltpu.make_async_copy(x_hbm.at[0, 0], bv, sem).wait()

    def do_unit(t, av, bv, sem_cur, av_n, bv_n, sem_next):
        bb, p, is_edge, _, _ = unit_rows(t)
        u1 = jnp.where(is_edge, 0, 2 * p + 1)
        u2 = jnp.where(is_edge, HD - 1, 2 * p + 2)

        is_lo = p == 0
        is_hi = p == UNITS_PER_IMG - 2

        def wsel(lo, hi, edge, default):
            r = jnp.where(is_lo, _splat(lo), _splat(default))
            r = jnp.where(is_hi, _splat(hi), r)
            return jnp.where(is_edge, _splat(edge), r)

        a1 = wsel(1.0, TWO_THIRD, THIRD, 0.75)
        b1 = wsel(0.0, THIRD, 0.0, 0.25)
        a2 = wsel(THIRD, 0.0, 0.0, 0.25)
        b2 = wsel(TWO_THIRD, 1.0, THIRD, 0.75)

        wait_in(av, bv, sem_cur)

        @pl.when(t + 1 < UPW)
        def _():
            start_in(t + 1, av_n, bv_n, sem_next)

        def hrow(w, c2):
            for cc in range(CV):
                s = pl.ds(cc * LANES, LANES)
                xa = av[w, s]
                xb = bv[w, s]
                av[w, s] = a1 * xa + b1 * xb
                bv[w, s] = a2 * xa + b2 * xb
            return c2

        lax.fori_loop(0, W, hrow, 0)

        @pl.when(t > 0)
        def _():
            pltpu.make_async_copy(out_hbm.at[0, 0], o1_v, out_sem1).wait()

        _col_mix(av, o1_v)
        pltpu.async_copy(o1_v, out_hbm.at[bb, u1], out_sem1)

        @pl.when(t > 0)
        def _():
            pltpu.make_async_copy(out_hbm.at[0, 0], o2_v, out_sem2).wait()

        _col_mix(bv, o2_v)
        pltpu.async_copy(o2_v, out_hbm.at[bb, u2], out_sem2)

    start_in(0, a0_v, b0_v, in_sem0)

    def pair(q, carry):
        do_unit(2 * q, a0_v, b0_v, in_sem0, a1_v, b1_v, in_sem1)
        do_unit(2 * q + 1, a1_v, b1_v, in_sem1, a0_v, b0_v, in_sem0)
        return carry

    lax.fori_loop(0, UPW // 2, pair, 0)

    pltpu.make_async_copy(out_hbm.at[0, 0], o1_v, out_sem1).wait()
    pltpu.make_async_copy(out_hbm.at[0, 0], o2_v, out_sem2).wait()


@jax.jit
def _upsample(x):
    mesh = plsc.VectorSubcoreMesh(core_axis_name="c", subcore_axis_name="s")
    f = functools.partial(
        pl.kernel,
        mesh=mesh,
        out_type=jax.ShapeDtypeStruct((B, HD, WD, C), jnp.float32),
        scratch_types=[
            pltpu.VMEM((W, C), jnp.float32),
            pltpu.VMEM((W, C), jnp.float32),
            pltpu.VMEM((W, C), jnp.float32),
            pltpu.VMEM((W, C), jnp.float32),
            pltpu.VMEM((WD, C), jnp.float32),
            pltpu.VMEM((WD, C), jnp.float32),
            pltpu.SemaphoreType.DMA,
            pltpu.SemaphoreType.DMA,
            pltpu.SemaphoreType.DMA,
            pltpu.SemaphoreType.DMA,
        ],
    )(_body)
    return f(x)


def kernel(inputs):
    return _upsample(inputs)
